# baseline (speedup 1.0000x reference)
import jax
import jax.numpy as jnp
from jax.experimental import pallas as pl
from jax.experimental.pallas import tpu as pltpu

_C = 4
_H = 32
_PATCH = 36
_TN = 16384


def _mlp_kernel(x_ref, w1_ref, w2_ref, o_ref):
    tn = x_ref.shape[-1]
    x = x_ref[...].reshape(_PATCH, tn)
    ones = jnp.ones((1, tn), jnp.float32)
    xa = jnp.concatenate([x, ones], axis=0)
    h = jax.lax.dot_general(
        w1_ref[: _PATCH + 1, : _H + 1], xa,
        (((0,), (0,)), ((), ())),
        preferred_element_type=jnp.float32,
    )
    h = jnp.maximum(h, 0.0)
    o_ref[...] = jax.lax.dot_general(
        w2_ref[: _H + 1, :_C], h,
        (((0,), (0,)), ((), ())),
        preferred_element_type=jnp.float32,
    )


def kernel(neighborhoods, w1_pad, w2_pad):
    n = neighborhoods.shape[0]
    xt = jnp.transpose(neighborhoods.astype(jnp.float32), (1, 2, 3, 0))
    xt = xt.reshape(9, 4, n)
    n_pad = pl.cdiv(n, _TN) * _TN
    if n_pad != n:
        xt = jnp.pad(xt, ((0, 0), (0, 0), (0, n_pad - n)))

    grid = n_pad // _TN
    inner = max(grid // 2, 1)
    out = pl.pallas_call(
        _mlp_kernel,
        out_shape=jax.ShapeDtypeStruct((_C, n_pad), jnp.float32),
        grid=(grid // inner, inner),
        in_specs=[
            pl.BlockSpec((9, 4, _TN), lambda i, j: (0, 0, i * inner + j)),
            pl.BlockSpec((128, 128), lambda i, j: (0, 0)),
            pl.BlockSpec((128, 128), lambda i, j: (0, 0)),
        ],
        out_specs=pl.BlockSpec((_C, _TN), lambda i, j: (0, i * inner + j)),
        compiler_params=pltpu.CompilerParams(
            dimension_semantics=("parallel", "arbitrary")),
    )(xt, w1_pad, w2_pad)
    return jnp.transpose(out[:, :n])


# speedup vs baseline: 1.2591x; 1.2591x over previous
import jax
import jax.numpy as jnp
from jax.experimental import pallas as pl
from jax.experimental.pallas import tpu as pltpu

_C = 4
_H = 32
_PATCH = 36
_TN = 32768


def _mlp_kernel(x_ref, w1_ref, w2_ref, o_ref):
    tn = x_ref.shape[-1]
    x = x_ref[...].reshape(_PATCH, tn)
    ones = jnp.ones((1, tn), jnp.float32)
    xa = jnp.concatenate([x, ones], axis=0)
    h = jax.lax.dot_general(
        w1_ref[: _PATCH + 1, : _H + 1], xa,
        (((0,), (0,)), ((), ())),
        preferred_element_type=jnp.float32,
    )
    h = jnp.maximum(h, 0.0)
    o_ref[...] = jax.lax.dot_general(
        w2_ref[: _H + 1, :_C], h,
        (((0,), (0,)), ((), ())),
        preferred_element_type=jnp.float32,
    )


def kernel(neighborhoods, w1_pad, w2_pad):
    n = neighborhoods.shape[0]
    xt = jnp.transpose(neighborhoods.astype(jnp.float32), (1, 2, 3, 0))
    xt = xt.reshape(9, 4, n)
    n_pad = pl.cdiv(n, _TN) * _TN
    if n_pad != n:
        xt = jnp.pad(xt, ((0, 0), (0, 0), (0, n_pad - n)))

    grid = n_pad // _TN
    inner = max(grid // 2, 1)
    out = pl.pallas_call(
        _mlp_kernel,
        out_shape=jax.ShapeDtypeStruct((_C, n_pad), jnp.float32),
        grid=(grid // inner, inner),
        in_specs=[
            pl.BlockSpec((9, 4, _TN), lambda i, j: (0, 0, i * inner + j)),
            pl.BlockSpec((128, 128), lambda i, j: (0, 0)),
            pl.BlockSpec((128, 128), lambda i, j: (0, 0)),
        ],
        out_specs=pl.BlockSpec((_C, _TN), lambda i, j: (0, i * inner + j)),
        compiler_params=pltpu.CompilerParams(
            dimension_semantics=("parallel", "arbitrary")),
    )(xt, w1_pad, w2_pad)
    return jnp.transpose(out[:, :n])


# speedup vs baseline: 1.3569x; 1.0777x over previous
import jax
import jax.numpy as jnp
from jax.experimental import pallas as pl
from jax.experimental.pallas import tpu as pltpu

_C = 4
_H = 32
_PATCH = 36
_TN = 65536


def _mlp_kernel(x_ref, w1_ref, w2_ref, o_ref):
    tn = x_ref.shape[-1]
    x = x_ref[...].reshape(_PATCH, tn)
    ones = jnp.ones((1, tn), jnp.float32)
    xa = jnp.concatenate([x, ones], axis=0)
    h = jax.lax.dot_general(
        w1_ref[: _PATCH + 1, : _H + 1], xa,
        (((0,), (0,)), ((), ())),
        preferred_element_type=jnp.float32,
    )
    h = jnp.maximum(h, 0.0)
    o_ref[...] = jax.lax.dot_general(
        w2_ref[: _H + 1, :_C], h,
        (((0,), (0,)), ((), ())),
        preferred_element_type=jnp.float32,
    )


def kernel(neighborhoods, w1_pad, w2_pad):
    n = neighborhoods.shape[0]
    xt = jnp.transpose(neighborhoods.astype(jnp.float32), (1, 2, 3, 0))
    xt = xt.reshape(9, 4, n)
    n_pad = pl.cdiv(n, _TN) * _TN
    if n_pad != n:
        xt = jnp.pad(xt, ((0, 0), (0, 0), (0, n_pad - n)))

    grid = n_pad // _TN
    out = pl.pallas_call(
        _mlp_kernel,
        out_shape=jax.ShapeDtypeStruct((_C, n_pad), jnp.float32),
        grid=(grid,),
        in_specs=[
            pl.BlockSpec((9, 4, _TN), lambda i: (0, 0, i)),
            pl.BlockSpec((128, 128), lambda i: (0, 0)),
            pl.BlockSpec((128, 128), lambda i: (0, 0)),
        ],
        out_specs=pl.BlockSpec((_C, _TN), lambda i: (0, i)),
        compiler_params=pltpu.CompilerParams(dimension_semantics=("parallel",)),
    )(xt, w1_pad, w2_pad)
    return jnp.transpose(out[:, :n])
